# Initial kernel scaffold; baseline (speedup 1.0000x reference)
import functools

import jax
import jax.numpy as jnp
from jax import lax
from jax.experimental import pallas as pl
from jax.experimental.pallas import tpu as pltpu
from jax.experimental.pallas import tpu_sc as plsc

N = 10000
E = 320000
D = 128
H = 64
G = 64

NW = 32
R = 313
NPAD = 10240
C = 2560
CG = 512
GCAP = CG + C + 16
NEG = float("-inf")


def _make_sc_scatter_max(F):
    mesh = plsc.VectorSubcoreMesh(core_axis_name="c", subcore_axis_name="s")
    fvecs = F // 16

    @functools.partial(
        pl.kernel,
        mesh=mesh,
        out_type=jax.ShapeDtypeStruct((NPAD, F), jnp.float32),
        scratch_types=[
            pltpu.VMEM((C,), jnp.int32),
            pltpu.VMEM((C,), jnp.int32),
            pltpu.VMEM((GCAP,), jnp.int32),
            pltpu.VMEM((GCAP,), jnp.int32),
            pltpu.VMEM((CG, F), jnp.float32),
            pltpu.VMEM((R + 1, F), jnp.float32),
            pltpu.SemaphoreType.DMA,
        ],
    )
    def sc_kernel(src_hbm, dst_hbm, x_hbm, out_hbm,
                  srcbuf, dstbuf, gsrc, gdl, rows, acc, sem):
        wid = lax.axis_index("s") * 2 + lax.axis_index("c")
        lo = wid * R

        neg = jnp.full((16,), NEG, dtype=jnp.float32)

        def init_row(r, carry):
            for f in range(fvecs):
                acc[r, pl.ds(16 * f, 16)] = neg
            return carry
        lax.fori_loop(0, R + 1, init_row, 0)

        def process_batch():
            pltpu.async_copy(x_hbm.at[gsrc.at[pl.ds(0, CG)]], rows, sem).wait()

            def proc(e, carry):
                dl = gdl[e]
                for f in range(fvecs):
                    sl = pl.ds(16 * f, 16)
                    acc[dl, sl] = jnp.maximum(acc[dl, sl], rows[e, sl])
                return carry
            lax.fori_loop(0, CG, proc, 0)

        def drain(fill):
            process_batch()
            rem = fill - CG
            nmove = (rem + 15) // 16

            def mv(i, carry):
                gsrc[pl.ds(16 * i, 16)] = gsrc[pl.ds(CG + 16 * i, 16)]
                gdl[pl.ds(16 * i, 16)] = gdl[pl.ds(CG + 16 * i, 16)]
                return carry
            lax.fori_loop(0, nmove, mv, 0)
            return rem

        def chunk_step(c, fill):
            pltpu.sync_copy(src_hbm.at[pl.ds(c * C, C)], srcbuf)
            pltpu.sync_copy(dst_hbm.at[pl.ds(c * C, C)], dstbuf)

            def filt(j, fl):
                d = dstbuf[pl.ds(16 * j, 16)]
                s = srcbuf[pl.ds(16 * j, 16)]
                m = (d >= lo) & (d < lo + R)
                plsc.store_compressed(gsrc.at[pl.ds(fl, 16)], s, mask=m)
                plsc.store_compressed(gdl.at[pl.ds(fl, 16)], d - lo, mask=m)
                return fl + jnp.max(plsc.all_reduce_population_count(m))
            fill = lax.fori_loop(0, C // 16, filt, fill)
            return lax.while_loop(lambda f: f >= CG, drain, fill)

        fill = lax.fori_loop(0, E // C, chunk_step, 0)

        pad_src = jnp.zeros((16,), dtype=jnp.int32)
        pad_dl = jnp.full((16,), R, dtype=jnp.int32)
        for p in range(CG // 16):
            gsrc[pl.ds(fill + 16 * p, 16)] = pad_src
            gdl[pl.ds(fill + 16 * p, 16)] = pad_dl
        process_batch()

        def fix_row(r, carry):
            for f in range(fvecs):
                sl = pl.ds(16 * f, 16)
                v = acc[r, sl]
                acc[r, sl] = jnp.where(v == NEG, 0.0, v)
            return carry
        lax.fori_loop(0, R, fix_row, 0)
        pltpu.sync_copy(acc.at[pl.ds(0, R)], out_hbm.at[pl.ds(wid * R, R)])

    return sc_kernel


_sc_scatter_max_d = _make_sc_scatter_max(D)
_sc_scatter_max_h = _make_sc_scatter_max(H)


def _tc_layer(agg, h, Wl, Wr, b):
    BN = 512
    npad, fa = agg.shape
    fh = h.shape[1]
    b2 = b.reshape(1, H)

    def body(agg_ref, h_ref, wl_ref, wr_ref, b_ref, o_ref):
        acc = jnp.dot(agg_ref[...], wl_ref[...],
                      preferred_element_type=jnp.float32)
        acc += jnp.dot(h_ref[...], wr_ref[...],
                       preferred_element_type=jnp.float32)
        o_ref[...] = jnp.maximum(acc + b_ref[...], 0.0)

    return pl.pallas_call(
        body,
        grid=(npad // BN,),
        in_specs=[
            pl.BlockSpec((BN, fa), lambda i: (i, 0)),
            pl.BlockSpec((BN, fh), lambda i: (i, 0)),
            pl.BlockSpec((fa, H), lambda i: (0, 0)),
            pl.BlockSpec((fh, H), lambda i: (0, 0)),
            pl.BlockSpec((1, H), lambda i: (0, 0)),
        ],
        out_specs=pl.BlockSpec((BN, H), lambda i: (i, 0)),
        out_shape=jax.ShapeDtypeStruct((npad, H), jnp.float32),
    )(agg, h, Wl, Wr, b2)


def _tc_pool_mlp(h3, batchp, A1, ab1, A2, ab2):
    BN = 512
    npad = h3.shape[0]
    ys = A2.shape[1]
    a1b = ab1.reshape(1, A1.shape[1])
    a2b = ab2.reshape(1, ys)

    def body(h_ref, b_ref, a1_ref, ab1_ref, a2_ref, ab2_ref, o_ref, acc_ref):
        i = pl.program_id(0)

        @pl.when(i == 0)
        def _():
            acc_ref[...] = jnp.full_like(acc_ref, NEG)

        hb = h_ref[...]
        ids = b_ref[...]
        parts = [
            jnp.max(jnp.where(ids == g, hb, NEG), axis=0, keepdims=True)
            for g in range(G)
        ]
        acc_ref[...] = jnp.maximum(acc_ref[...], jnp.concatenate(parts, 0))

        @pl.when(i == pl.num_programs(0) - 1)
        def _():
            pooled = acc_ref[...]
            pooled = jnp.where(pooled == NEG, 0.0, pooled)
            t = jnp.dot(pooled, a1_ref[...], preferred_element_type=jnp.float32)
            t = jnp.maximum(t + ab1_ref[...], 0.0)
            o_ref[...] = jnp.dot(t, a2_ref[...],
                                 preferred_element_type=jnp.float32) + ab2_ref[...]

    return pl.pallas_call(
        body,
        grid=(npad // BN,),
        in_specs=[
            pl.BlockSpec((BN, H), lambda i: (i, 0)),
            pl.BlockSpec((BN, 1), lambda i: (i, 0)),
            pl.BlockSpec(A1.shape, lambda i: (0, 0)),
            pl.BlockSpec((1, A1.shape[1]), lambda i: (0, 0)),
            pl.BlockSpec(A2.shape, lambda i: (0, 0)),
            pl.BlockSpec((1, ys), lambda i: (0, 0)),
        ],
        out_specs=pl.BlockSpec((G, ys), lambda i: (0, 0)),
        out_shape=jax.ShapeDtypeStruct((G, ys), jnp.float32),
        scratch_shapes=[pltpu.VMEM((G, H), jnp.float32)],
    )(h3, batchp, A1, a1b, A2, a2b)


def kernel(x, edge_index, batch, W1l, W1r, b1, W2l, W2r, b2, W3l, W3r, b3,
           A1, ab1, A2, ab2):
    src = edge_index[0]
    dst = edge_index[1]
    x_p = jnp.zeros((NPAD, D), jnp.float32).at[:N].set(x)
    batchp = jnp.concatenate(
        [batch, jnp.full((NPAD - N,), G, jnp.int32)]).reshape(NPAD, 1)

    agg1 = _sc_scatter_max_d(src, dst, x_p)
    h1 = _tc_layer(agg1, x_p, W1l, W1r, b1)
    agg2 = _sc_scatter_max_h(src, dst, h1)
    h2 = _tc_layer(agg2, h1, W2l, W2r, b2)
    agg3 = _sc_scatter_max_h(src, dst, h2)
    h3 = _tc_layer(agg3, h2, W3l, W3r, b3)
    return _tc_pool_mlp(h3, batchp, A1, ab1, A2, ab2)


# speedup vs baseline: 1.7263x; 1.7263x over previous
import functools

import jax
import jax.numpy as jnp
from jax import lax
from jax.experimental import pallas as pl
from jax.experimental.pallas import tpu as pltpu
from jax.experimental.pallas import tpu_sc as plsc

N = 10000
E = 320000
D = 128
H = 64
G = 64

NW = 32
R = 320
NPAD = 10240
C = 2560
CG = 512
GCAP = CG + C + 16
TRASH = GCAP
GBUF = GCAP + 16
NEG = float("-inf")


def _make_sc_scatter_max(F):
    mesh = plsc.VectorSubcoreMesh(core_axis_name="c", subcore_axis_name="s")
    fvecs = F // 16

    @functools.partial(
        pl.kernel,
        mesh=mesh,
        out_type=jax.ShapeDtypeStruct((NPAD, F), jnp.float32),
        compiler_params=pltpu.CompilerParams(
            needs_layout_passes=False, use_tc_tiling_on_sc=False),
        scratch_types=[
            pltpu.VMEM((C,), jnp.int32),
            pltpu.VMEM((C,), jnp.int32),
            pltpu.VMEM((GBUF,), jnp.int32),
            pltpu.VMEM((GBUF,), jnp.int32),
            pltpu.VMEM((CG, F), jnp.float32),
            pltpu.VMEM((R + 1, F), jnp.float32),
            pltpu.SemaphoreType.DMA,
        ],
    )
    def sc_kernel(src_hbm, dst_hbm, x_hbm, out_hbm,
                  srcbuf, dstbuf, gsrc, gdl, rows, acc, sem):
        wid = lax.axis_index("s") * 2 + lax.axis_index("c")
        lo = wid * R

        neg = jnp.full((16,), NEG, dtype=jnp.float32)

        def init_row(r, carry):
            for f in range(fvecs):
                acc[r, pl.ds(16 * f, 16)] = neg
            return carry
        lax.fori_loop(0, R + 1, init_row, 0)

        def process_batch():
            pltpu.async_copy(x_hbm.at[gsrc.at[pl.ds(0, CG)]], rows, sem).wait()

            def proc(eg, carry):
                dls = gdl[pl.ds(16 * eg, 16)]
                for k in range(16):
                    dl = dls[k]
                    for f in range(fvecs):
                        sl = pl.ds(16 * f, 16)
                        acc[dl, sl] = jnp.maximum(acc[dl, sl],
                                                  rows[16 * eg + k, sl])
                return carry
            lax.fori_loop(0, CG // 16, proc, 0)

        def drain(fill):
            process_batch()
            rem = fill - CG
            nmove = (rem + 15) // 16

            def mv(i, carry):
                gsrc[pl.ds(16 * i, 16)] = gsrc[pl.ds(CG + 16 * i, 16)]
                gdl[pl.ds(16 * i, 16)] = gdl[pl.ds(CG + 16 * i, 16)]
                return carry
            lax.fori_loop(0, nmove, mv, 0)
            return rem

        def chunk_step(c, fill):
            pltpu.sync_copy(src_hbm.at[pl.ds(c * C, C)], srcbuf)
            pltpu.sync_copy(dst_hbm.at[pl.ds(c * C, C)], dstbuf)

            def filt(j, fl):
                d = dstbuf[pl.ds(16 * j, 16)]
                s = srcbuf[pl.ds(16 * j, 16)]
                m = (d >= lo) & (d < lo + R)
                pos = plsc.cumsum(jnp.where(m, 1, 0))
                idx = jnp.where(m, fl + pos - 1, TRASH)
                plsc.store_scatter(gsrc, [idx], s)
                plsc.store_scatter(gdl, [idx], jnp.where(m, d - lo, R))
                return fl + pos[15]
            fill = lax.fori_loop(0, C // 16, filt, fill)
            return lax.while_loop(lambda f: f >= CG, drain, fill)

        fill = lax.fori_loop(0, E // C, chunk_step, 0)

        pad_src = jnp.zeros((16,), dtype=jnp.int32)
        pad_dl = jnp.full((16,), R, dtype=jnp.int32)
        for p in range(CG // 16):
            gsrc[pl.ds(fill + 16 * p, 16)] = pad_src
            gdl[pl.ds(fill + 16 * p, 16)] = pad_dl
        process_batch()

        def fix_row(r, carry):
            for f in range(fvecs):
                sl = pl.ds(16 * f, 16)
                v = acc[r, sl]
                acc[r, sl] = jnp.where(v == NEG, 0.0, v)
            return carry
        lax.fori_loop(0, R, fix_row, 0)
        pltpu.sync_copy(acc.at[pl.ds(0, R)], out_hbm.at[pl.ds(wid * R, R)])

    return sc_kernel


_sc_scatter_max_d = _make_sc_scatter_max(D)
_sc_scatter_max_h = _make_sc_scatter_max(H)


def _tc_layer(agg, h, Wl, Wr, b):
    BN = 512
    npad, fa = agg.shape
    fh = h.shape[1]
    b2 = b.reshape(1, H)

    def body(agg_ref, h_ref, wl_ref, wr_ref, b_ref, o_ref):
        acc = jnp.dot(agg_ref[...], wl_ref[...],
                      preferred_element_type=jnp.float32)
        acc += jnp.dot(h_ref[...], wr_ref[...],
                       preferred_element_type=jnp.float32)
        o_ref[...] = jnp.maximum(acc + b_ref[...], 0.0)

    return pl.pallas_call(
        body,
        grid=(npad // BN,),
        in_specs=[
            pl.BlockSpec((BN, fa), lambda i: (i, 0)),
            pl.BlockSpec((BN, fh), lambda i: (i, 0)),
            pl.BlockSpec((fa, H), lambda i: (0, 0)),
            pl.BlockSpec((fh, H), lambda i: (0, 0)),
            pl.BlockSpec((1, H), lambda i: (0, 0)),
        ],
        out_specs=pl.BlockSpec((BN, H), lambda i: (i, 0)),
        out_shape=jax.ShapeDtypeStruct((npad, H), jnp.float32),
    )(agg, h, Wl, Wr, b2)


def _tc_pool_mlp(h3, batchp, A1, ab1, A2, ab2):
    BN = 512
    npad = h3.shape[0]
    ys = A2.shape[1]
    a1b = ab1.reshape(1, A1.shape[1])
    a2b = ab2.reshape(1, ys)

    def body(h_ref, b_ref, a1_ref, ab1_ref, a2_ref, ab2_ref, o_ref, acc_ref):
        i = pl.program_id(0)

        @pl.when(i == 0)
        def _():
            acc_ref[...] = jnp.full_like(acc_ref, NEG)

        hb = h_ref[...]
        ids = b_ref[...]
        parts = [
            jnp.max(jnp.where(ids == g, hb, NEG), axis=0, keepdims=True)
            for g in range(G)
        ]
        acc_ref[...] = jnp.maximum(acc_ref[...], jnp.concatenate(parts, 0))

        @pl.when(i == pl.num_programs(0) - 1)
        def _():
            pooled = acc_ref[...]
            pooled = jnp.where(pooled == NEG, 0.0, pooled)
            t = jnp.dot(pooled, a1_ref[...], preferred_element_type=jnp.float32)
            t = jnp.maximum(t + ab1_ref[...], 0.0)
            o_ref[...] = jnp.dot(t, a2_ref[...],
                                 preferred_element_type=jnp.float32) + ab2_ref[...]

    return pl.pallas_call(
        body,
        grid=(npad // BN,),
        in_specs=[
            pl.BlockSpec((BN, H), lambda i: (i, 0)),
            pl.BlockSpec((BN, 1), lambda i: (i, 0)),
            pl.BlockSpec(A1.shape, lambda i: (0, 0)),
            pl.BlockSpec((1, A1.shape[1]), lambda i: (0, 0)),
            pl.BlockSpec(A2.shape, lambda i: (0, 0)),
            pl.BlockSpec((1, ys), lambda i: (0, 0)),
        ],
        out_specs=pl.BlockSpec((G, ys), lambda i: (0, 0)),
        out_shape=jax.ShapeDtypeStruct((G, ys), jnp.float32),
        scratch_shapes=[pltpu.VMEM((G, H), jnp.float32)],
    )(h3, batchp, A1, a1b, A2, a2b)


def kernel(x, edge_index, batch, W1l, W1r, b1, W2l, W2r, b2, W3l, W3r, b3,
           A1, ab1, A2, ab2):
    src = edge_index[0]
    dst = edge_index[1]
    x_p = jnp.zeros((NPAD, D), jnp.float32).at[:N].set(x)
    batchp = jnp.concatenate(
        [batch, jnp.full((NPAD - N,), G, jnp.int32)]).reshape(NPAD, 1)

    agg1 = _sc_scatter_max_d(src, dst, x_p)
    h1 = _tc_layer(agg1, x_p, W1l, W1r, b1)
    agg2 = _sc_scatter_max_h(src, dst, h1)
    h2 = _tc_layer(agg2, h1, W2l, W2r, b2)
    agg3 = _sc_scatter_max_h(src, dst, h2)
    h3 = _tc_layer(agg3, h2, W3l, W3r, b3)
    return _tc_pool_mlp(h3, batchp, A1, ab1, A2, ab2)
